# baseline (speedup 1.0000x reference)
import functools

import jax
import jax.numpy as jnp
from jax import lax
from jax.experimental import pallas as pl
from jax.experimental.pallas import tpu as pltpu
from jax.experimental.pallas import tpu_sc as plsc

B = 8
P = 8
N = 8192
G = 64
V = G ** 3
CHUNKS = N // 16
PAIRS_PER_WORKER = (B * P) // 32


def _sc_body(px_hbm, py_hbm, pz_hbm, planes_hbm, t1_hbm, t2_hbm, out_hbm,
             px_v, py_v, pz_v,
             iv0_v, iv1_v,
             ga0_v, gb0_v, ga1_v, gb1_v,
             plane_v, acc_v,
             sem_a0, sem_b0, sem_a1, sem_b1):
    wid = lax.axis_index("s") * 2 + lax.axis_index("c")
    b = wid // 4

    pbase = b * N
    pltpu.sync_copy(px_hbm.at[pl.ds(pl.multiple_of(pbase, N), N)], px_v)
    pltpu.sync_copy(py_hbm.at[pl.ds(pl.multiple_of(pbase, N), N)], py_v)
    pltpu.sync_copy(pz_hbm.at[pl.ds(pl.multiple_of(pbase, N), N)], pz_v)

    base_off = b * V

    def load_plane(k):
        pair = wid * PAIRS_PER_WORKER + k
        pltpu.sync_copy(
            planes_hbm.at[pl.ds(pl.multiple_of(pair * 64, 64), 64)], plane_v)
        nx = plane_v[pl.ds(0, 16)]
        ny = plane_v[pl.ds(16, 16)]
        nz = plane_v[pl.ds(32, 16)]
        dd = plane_v[pl.ds(48, 16)]
        inv2 = 2.0 / (nx * nx + ny * ny + nz * nz)
        return nx, ny, nz, dd, inv2

    def reflect(sl, pp):
        nx, ny, nz, dd, inv2 = pp
        px = px_v[sl]
        py = py_v[sl]
        pz = pz_v[sl]
        f = (px * nx + py * ny + pz * nz + dd) * inv2
        return px - f * nx, py - f * ny, pz - f * nz

    def stage_a(pp, iv_v):
        def body_a(r, carry):
            sl = pl.ds(pl.multiple_of(r * 16, 16), 16)
            tx, ty, tz = reflect(sl, pp)

            def ceil_i(t):
                z = (t + 0.5) * float(G) - 0.5
                i = z.astype(jnp.int32)
                return jnp.where(z > i.astype(jnp.float32), i + 1, i)

            flat = ceil_i(tx) * (G * G) + ceil_i(ty) * G + ceil_i(tz)
            flat = jnp.minimum(jnp.maximum(flat, 0), V - 1)
            iv_v[sl] = flat + base_off
            return carry

        lax.fori_loop(0, CHUNKS, body_a, 0, unroll=8)

    def fire_gathers(iv_v, ga_v, gb_v, sems):
        cps = []
        for h in range(4):
            hs = pl.ds(h * (N // 4), N // 4)
            cps.append(pltpu.async_copy(t1_hbm.at[iv_v.at[hs]], ga_v.at[hs], sems[0]))
            cps.append(pltpu.async_copy(t2_hbm.at[iv_v.at[hs]], gb_v.at[hs], sems[1]))
        return cps

    def stage_c(pp, ga_v, gb_v, acc):
        def body_c(r, a):
            sl = pl.ds(pl.multiple_of(r * 16, 16), 16)
            tx, ty, tz = reflect(sl, pp)
            w1 = ga_v[sl]
            w2 = gb_v[sl]
            cx = lax.bitcast_convert_type(w1 << 16, jnp.float32)
            cy = lax.bitcast_convert_type(w1 & -65536, jnp.float32)
            cz = lax.bitcast_convert_type(w2 << 16, jnp.float32)
            m2 = lax.bitcast_convert_type(w2 & -65536, jnp.float32)
            dx = tx - cx
            dy = ty - cy
            dz = tz - cz
            return a + m2 * (dx * dx + dy * dy + dz * dz)

        return lax.fori_loop(0, CHUNKS, body_c, acc, unroll=8)

    pp0 = load_plane(0)
    stage_a(pp0, iv0_v)
    cps0 = fire_gathers(iv0_v, ga0_v, gb0_v, (sem_a0, sem_b0))
    pp1 = load_plane(1)
    stage_a(pp1, iv1_v)
    for cp in cps0:
        cp.wait()
    cps1 = fire_gathers(iv1_v, ga1_v, gb1_v, (sem_a1, sem_b1))
    acc = stage_c(pp0, ga0_v, gb0_v, jnp.zeros((16,), jnp.float32))
    for cp in cps1:
        cp.wait()
    acc = stage_c(pp1, ga1_v, gb1_v, acc)

    acc_v[...] = acc
    pltpu.sync_copy(acc_v, out_hbm.at[pl.ds(pl.multiple_of(wid * 16, 16), 16)])


@jax.jit
def _sc_loss(px, py, pz, planes_pad, t1, t2):
    mesh = plsc.VectorSubcoreMesh(core_axis_name="c", subcore_axis_name="s")
    f32 = jnp.float32
    i32 = jnp.int32
    kern = functools.partial(
        pl.kernel,
        mesh=mesh,
        out_type=jax.ShapeDtypeStruct((32 * 16,), f32),
        scratch_types=(
            [pltpu.VMEM((N,), f32) for _ in range(3)]
            + [pltpu.VMEM((N,), i32) for _ in range(2)]
            + [pltpu.VMEM((N,), i32) for _ in range(4)]
            + [pltpu.VMEM((64,), f32)]
            + [pltpu.VMEM((16,), f32)]
            + [pltpu.SemaphoreType.DMA for _ in range(4)]
        ),
    )(_sc_body)
    return kern(px, py, pz, planes_pad, t1, t2)


def _pack_pair(lo, hi):
    lo16 = jax.lax.bitcast_convert_type(
        lo.astype(jnp.bfloat16), jnp.uint16).astype(jnp.uint32)
    hi16 = jax.lax.bitcast_convert_type(
        hi.astype(jnp.bfloat16), jnp.uint16).astype(jnp.uint32)
    return jax.lax.bitcast_convert_type(lo16 | (hi16 << 16), jnp.int32)


def kernel(voxel, points, closest_points, planes):
    s = 1.0 + 0.0 * jnp.sum(planes)
    px = (points[:, :, 0] * s).reshape(-1)
    py = (points[:, :, 1] * s).reshape(-1)
    pz = (points[:, :, 2] * s).reshape(-1)
    cpy = closest_points[:, :, 1] * s
    cpx = closest_points[:, :, 0] * s
    cpz = closest_points[:, :, 2] * s
    mask = 1.0 - voxel.reshape(B, V)
    m2 = mask * mask
    t1 = _pack_pair(cpx, cpy).reshape(-1)
    t2 = _pack_pair(cpz, m2).reshape(-1)
    planes_pad = (jnp.broadcast_to(
        planes.reshape(B * P, 4)[:, :, None], (B * P, 4, 16)) * s).reshape(-1)
    partial = _sc_loss(px, py, pz, planes_pad, t1, t2)
    return jnp.sum(partial) / (B * P)


# speedup vs baseline: 1.0288x; 1.0288x over previous
import functools

import jax
import jax.numpy as jnp
from jax import lax
from jax.experimental import pallas as pl
from jax.experimental.pallas import tpu as pltpu
from jax.experimental.pallas import tpu_sc as plsc

B = 8
P = 8
N = 8192
G = 64
V = G ** 3
CHUNKS = N // 16
PAIRS_PER_WORKER = (B * P) // 32


def _sc_body(px_hbm, py_hbm, pz_hbm, planes_hbm, cpx_hbm, cpy_hbm, cpz_hbm,
             m2_hbm, out_hbm,
             px_v, py_v, pz_v, tx_v, ty_v, tz_v,
             iv_v, ga_v, gb_v, gc_v, gv_v,
             plane_v, acc_v, sem_a, sem_b, sem_c, sem_v):
    wid = lax.axis_index("s") * 2 + lax.axis_index("c")
    b = wid // 4

    pbase = b * N
    pltpu.sync_copy(px_hbm.at[pl.ds(pl.multiple_of(pbase, N), N)], px_v)
    pltpu.sync_copy(py_hbm.at[pl.ds(pl.multiple_of(pbase, N), N)], py_v)
    pltpu.sync_copy(pz_hbm.at[pl.ds(pl.multiple_of(pbase, N), N)], pz_v)

    acc = jnp.zeros((16,), jnp.float32)
    base_off = b * V

    for k in range(PAIRS_PER_WORKER):
        pair = wid * PAIRS_PER_WORKER + k
        pltpu.sync_copy(
            planes_hbm.at[pl.ds(pl.multiple_of(pair * 64, 64), 64)], plane_v)
        nx = plane_v[pl.ds(0, 16)]
        ny = plane_v[pl.ds(16, 16)]
        nz = plane_v[pl.ds(32, 16)]
        dd = plane_v[pl.ds(48, 16)]
        inv2 = 2.0 / (nx * nx + ny * ny + nz * nz)

        def body_a(r, carry):
            sl = pl.ds(pl.multiple_of(r * 16, 16), 16)
            px = px_v[sl]
            py = py_v[sl]
            pz = pz_v[sl]
            f = (px * nx + py * ny + pz * nz + dd) * inv2
            tx = px - f * nx
            ty = py - f * ny
            tz = pz - f * nz
            tx_v[sl] = tx
            ty_v[sl] = ty
            tz_v[sl] = tz

            def ceil_i(t):
                z = (t + 0.5) * float(G) - 0.5
                i = z.astype(jnp.int32)
                return jnp.where(z > i.astype(jnp.float32), i + 1, i)

            flat = ceil_i(tx) * (G * G) + ceil_i(ty) * G + ceil_i(tz)
            flat = jnp.minimum(jnp.maximum(flat, 0), V - 1)
            iv_v[sl] = flat + base_off
            return carry

        lax.fori_loop(0, CHUNKS, body_a, 0, unroll=8)

        copies = []
        nsplit = 4
        csz = N // nsplit
        for h in range(nsplit):
            hs = pl.ds(h * csz, csz)
            copies.append(pltpu.async_copy(cpx_hbm.at[iv_v.at[hs]], ga_v.at[hs], sem_a))
            copies.append(pltpu.async_copy(cpy_hbm.at[iv_v.at[hs]], gb_v.at[hs], sem_b))
            copies.append(pltpu.async_copy(cpz_hbm.at[iv_v.at[hs]], gc_v.at[hs], sem_c))
            copies.append(pltpu.async_copy(m2_hbm.at[iv_v.at[hs]], gv_v.at[hs], sem_v))
        for cp in copies:
            cp.wait()

        def body_c(r, a):
            sl = pl.ds(pl.multiple_of(r * 16, 16), 16)
            dx = tx_v[sl] - ga_v[sl]
            dy = ty_v[sl] - gb_v[sl]
            dz = tz_v[sl] - gc_v[sl]
            return a + gv_v[sl] * (dx * dx + dy * dy + dz * dz)

        acc = lax.fori_loop(0, CHUNKS, body_c, acc, unroll=8)

    acc_v[...] = acc
    pltpu.sync_copy(acc_v, out_hbm.at[pl.ds(pl.multiple_of(wid * 16, 16), 16)])


@jax.jit
def _sc_loss(px, py, pz, planes_pad, cpx, cpy, cpz, m2):
    mesh = plsc.VectorSubcoreMesh(core_axis_name="c", subcore_axis_name="s")
    f32 = jnp.float32
    i32 = jnp.int32
    kern = functools.partial(
        pl.kernel,
        mesh=mesh,
        out_type=jax.ShapeDtypeStruct((32 * 16,), f32),
        scratch_types=[
            pltpu.VMEM((N,), f32),
            pltpu.VMEM((N,), f32),
            pltpu.VMEM((N,), f32),
            pltpu.VMEM((N,), f32),
            pltpu.VMEM((N,), f32),
            pltpu.VMEM((N,), f32),
            pltpu.VMEM((N,), i32),
            pltpu.VMEM((N,), f32),
            pltpu.VMEM((N,), f32),
            pltpu.VMEM((N,), f32),
            pltpu.VMEM((N,), f32),
            pltpu.VMEM((64,), f32),
            pltpu.VMEM((16,), f32),
            pltpu.SemaphoreType.DMA,
            pltpu.SemaphoreType.DMA,
            pltpu.SemaphoreType.DMA,
            pltpu.SemaphoreType.DMA,
        ],
    )(_sc_body)
    return kern(px, py, pz, planes_pad, cpx, cpy, cpz, m2)


def kernel(voxel, points, closest_points, planes):
    s = 1.0 + 0.0 * jnp.sum(planes)
    px = (points[:, :, 0] * s).reshape(-1)
    py = (points[:, :, 1] * s).reshape(-1)
    pz = (points[:, :, 2] * s).reshape(-1)
    cpx = (closest_points[:, :, 0] * s).reshape(-1)
    cpy = (closest_points[:, :, 1] * s).reshape(-1)
    cpz = (closest_points[:, :, 2] * s).reshape(-1)
    mask = 1.0 - voxel
    m2 = (mask * mask).reshape(-1)
    planes_pad = (jnp.broadcast_to(
        planes.reshape(B * P, 4)[:, :, None], (B * P, 4, 16)) * s).reshape(-1)
    partial = _sc_loss(px, py, pz, planes_pad, cpx, cpy, cpz, m2)
    return jnp.sum(partial) / (B * P)
